# Initial kernel scaffold; baseline (speedup 1.0000x reference)
import jax
import jax.numpy as jnp
from jax.experimental import pallas as pl


def kernel(obs, W_nn, b_nn):
    raise NotImplementedError("write your pallas kernel here")


# speedup vs baseline: 7.2590x; 7.2590x over previous
import jax
import jax.numpy as jnp
from jax.experimental import pallas as pl
from jax.experimental.pallas import tpu as pltpu

_H = 4
_W = 4
_K = 20
_NUM_OUT = 5
_C = _H * _W
_LANES = 128
_S = 8

_DIRS = [(-1, 0), (1, 0), (0, -1), (0, 1)]
_NBRS = []
for _i in range(_H):
    for _j in range(_W):
        _lst = []
        for _di, _dj in _DIRS:
            _ni, _nj = _i + _di, _j + _dj
            if 0 <= _ni < _H and 0 <= _nj < _W:
                _lst.append(_ni * _W + _nj)
        _NBRS.append(_lst)


def _vin_body(obs_ref, w_ref, b_ref, out_ref):
    p = [1.0 - obs_ref[3 * c] for c in range(_C)]
    rout = [obs_ref[3 * c + 2] for c in range(_C)]
    base = {}
    for c in range(_C):
        for nb in _NBRS[c]:
            base[(c, nb)] = rout[nb] - rout[c] - 0.05

    v = [jnp.zeros_like(p[0]) for _ in range(_C)]
    for _ in range(_K):
        new_v = []
        for c in range(_C):
            best = v[c]
            for nb in _NBRS[c]:
                best = jnp.maximum(best, p[c] * v[nb] + base[(c, nb)])
            new_v.append(best)
        v = new_v

    for n in range(_NUM_OUT):
        acc = jnp.full_like(p[0], b_ref[n])
        for c in range(_C):
            acc = acc + p[c] * w_ref[4 * c, n]
            acc = acc + obs_ref[3 * c + 1] * w_ref[4 * c + 1, n]
            acc = acc + v[c] * (w_ref[4 * c + 2, n] + w_ref[4 * c + 3, n])
        out_ref[n] = acc


def kernel(obs, W_nn, b_nn):
    B_, F = obs.shape
    chunk = _S * _LANES
    n_chunks = B_ // chunk
    obs_t = obs.T.reshape(F, B_ // _LANES, _LANES)
    out = pl.pallas_call(
        _vin_body,
        grid=(n_chunks,),
        in_specs=[
            pl.BlockSpec((F, _S, _LANES), lambda i: (0, i, 0)),
            pl.BlockSpec(memory_space=pltpu.SMEM),
            pl.BlockSpec(memory_space=pltpu.SMEM),
        ],
        out_specs=pl.BlockSpec((_NUM_OUT, _S, _LANES), lambda i: (0, i, 0)),
        out_shape=jax.ShapeDtypeStruct((_NUM_OUT, B_ // _LANES, _LANES), jnp.float32),
        compiler_params=pltpu.CompilerParams(dimension_semantics=("parallel",)),
    )(obs_t, W_nn, b_nn)
    return out.reshape(_NUM_OUT, B_).T


# speedup vs baseline: 10.3098x; 1.4203x over previous
import jax
import jax.numpy as jnp
from jax.experimental import pallas as pl
from jax.experimental.pallas import tpu as pltpu

_H = 4
_W = 4
_K = 20
_NUM_OUT = 5
_C = _H * _W
_LANES = 128
_S = 32

_DIRS = [(-1, 0), (1, 0), (0, -1), (0, 1)]
_NBRS = []
for _i in range(_H):
    for _j in range(_W):
        _lst = []
        for _di, _dj in _DIRS:
            _ni, _nj = _i + _di, _j + _dj
            if 0 <= _ni < _H and 0 <= _nj < _W:
                _lst.append(_ni * _W + _nj)
        _NBRS.append(_lst)


def _vin_body(obs_ref, w_ref, b_ref, out_ref):
    p = [1.0 - obs_ref[3 * c] for c in range(_C)]
    rout = [obs_ref[3 * c + 2] for c in range(_C)]
    base = {}
    for c in range(_C):
        for nb in _NBRS[c]:
            base[(c, nb)] = rout[nb] - rout[c] - 0.05

    v = []
    for c in range(_C):
        m = base[(c, _NBRS[c][0])]
        for nb in _NBRS[c][1:]:
            m = jnp.maximum(m, base[(c, nb)])
        v.append(jnp.maximum(m, 0.0))

    for _ in range(_K - 1):
        new_v = []
        for c in range(_C):
            best = v[c]
            for nb in _NBRS[c]:
                best = jnp.maximum(best, p[c] * v[nb] + base[(c, nb)])
            new_v.append(best)
        v = new_v

    for n in range(_NUM_OUT):
        acc = v[0] * (w_ref[2, n] + w_ref[3, n]) + b_ref[n]
        for c in range(1, _C):
            acc = acc + v[c] * (w_ref[4 * c + 2, n] + w_ref[4 * c + 3, n])
        for c in range(_C):
            acc = acc + p[c] * w_ref[4 * c, n]
            acc = acc + obs_ref[3 * c + 1] * w_ref[4 * c + 1, n]
        out_ref[n] = acc


def kernel(obs, W_nn, b_nn):
    B_, F = obs.shape
    chunk = _S * _LANES
    n_chunks = B_ // chunk
    obs_t = obs.T.reshape(F, B_ // _LANES, _LANES)
    out = pl.pallas_call(
        _vin_body,
        grid=(n_chunks,),
        in_specs=[
            pl.BlockSpec((F, _S, _LANES), lambda i: (0, i, 0)),
            pl.BlockSpec(memory_space=pltpu.SMEM),
            pl.BlockSpec(memory_space=pltpu.SMEM),
        ],
        out_specs=pl.BlockSpec((_NUM_OUT, _S, _LANES), lambda i: (0, i, 0)),
        out_shape=jax.ShapeDtypeStruct((_NUM_OUT, B_ // _LANES, _LANES), jnp.float32),
        compiler_params=pltpu.CompilerParams(dimension_semantics=("parallel",)),
    )(obs_t, W_nn, b_nn)
    return out.reshape(_NUM_OUT, B_).T
